# baseline (speedup 1.0000x reference)
import jax
import jax.numpy as jnp
from jax import lax
from jax.experimental import pallas as pl
from jax.experimental.pallas import tpu as pltpu
from jax.experimental.pallas import tpu_sc as plsc

N = 10000
E = 160000
D = 256
DH = 128
NC = 2
NS = 16
CH = 96
EPT = 10080
PAD = EPT - E // NS
EPAD = EPT * NS
NCHUNK = EPT // CH
ZROW = 2 * N
ACH = 80
NZ = N // ACH
ZK = (NZ + NS - 1) // NS


DEPTH = 4


def _segsum_body(gidx_hbm, dst_hbm, z_hbm, zeros_hbm, out_hbm, *rest):
    idxv = rest[0:4]
    dstv = rest[4:8]
    rowsv = rest[8:12]
    acc_sh = rest[12]
    sem_i = rest[13:17]
    sem_d = rest[17:21]
    sem_g = rest[21:25]
    sem_s = rest[25:29]

    c = lax.axis_index("c")
    s = lax.axis_index("s")

    stgv = rowsv[0].at[pl.ds(0, ACH)]
    pltpu.sync_copy(zeros_hbm, stgv)
    for k in range(ZK):
        cid = s + k * NS

        @pl.when(cid < NZ)
        def _():
            pltpu.sync_copy(stgv, acc_sh.at[pl.ds(cid * ACH, ACH)])
    plsc.subcore_barrier()

    def fire_idx(k, b):
        base = pl.multiple_of(s * EPT + k * CH, 8)
        pltpu.async_copy(gidx_hbm.at[pl.ds(c * EPAD + base, CH)], idxv[b],
                         sem_i[b])
        pltpu.async_copy(dst_hbm.at[pl.ds(base, CH)], dstv[b], sem_d[b])

    def wait_idx(b):
        pltpu.make_async_copy(gidx_hbm.at[pl.ds(0, CH)], idxv[b],
                              sem_i[b]).wait()
        pltpu.make_async_copy(dst_hbm.at[pl.ds(0, CH)], dstv[b],
                              sem_d[b]).wait()

    def fire_gather(b):
        pltpu.async_copy(z_hbm.at[idxv[b]], rowsv[b], sem_g[b])

    def wait_gather(b):
        pltpu.make_async_copy(z_hbm.at[idxv[b]], rowsv[b], sem_g[b]).wait()

    def fire_scatter(b):
        pltpu.async_copy(rowsv[b], acc_sh.at[dstv[b]], sem_s[b], add=True)

    def wait_scatter(b):
        pltpu.make_async_copy(rowsv[b], acc_sh.at[dstv[b]],
                              sem_s[b]).wait()

    fire_idx(0, 0)
    fire_idx(1, 1)
    wait_idx(0)
    fire_gather(0)

    @pl.loop(0, NCHUNK, step=DEPTH)
    def _(g):
        for b in range(DEPTH):
            k = g + b
            b1 = (b + 1) % DEPTH
            b2 = (b + 2) % DEPTH

            @pl.when(k < NCHUNK)
            def _():
                @pl.when(k + 2 < NCHUNK)
                def _():
                    @pl.when(k >= 2)
                    def _():
                        wait_scatter(b2)
                    fire_idx(k + 2, b2)

                @pl.when(k + 1 < NCHUNK)
                def _():
                    wait_idx(b1)
                    fire_gather(b1)
                wait_gather(b)
                fire_scatter(b)

    for j in range(DEPTH):
        wait_scatter((NCHUNK - DEPTH + j) % DEPTH)
    plsc.subcore_barrier()

    for k in range(ZK):
        cid = s + k * NS

        @pl.when(cid < NZ)
        def _():
            pltpu.sync_copy(acc_sh.at[pl.ds(cid * ACH, ACH)], stgv)
            pltpu.sync_copy(stgv, out_hbm.at[pl.ds(c * N + cid * ACH, ACH)])


def _segment_sum(gidx, dstp, z_pad):
    mesh = plsc.VectorSubcoreMesh(core_axis_name="c", subcore_axis_name="s")
    zeros = jnp.zeros((ACH, DH), jnp.float32)
    kern = pl.kernel(
        _segsum_body,
        out_type=jax.ShapeDtypeStruct((2 * N, DH), jnp.float32),
        mesh=mesh,
        scratch_types=(
            [pltpu.VMEM((CH,), jnp.int32)] * DEPTH
            + [pltpu.VMEM((CH,), jnp.int32)] * DEPTH
            + [pltpu.VMEM((CH, DH), jnp.float32)] * DEPTH
            + [pltpu.VMEM_SHARED((N, DH), jnp.float32)]
            + [pltpu.SemaphoreType.DMA] * (4 * DEPTH)
        ),
    )
    return kern(gidx, dstp, z_pad, zeros)


CHD = 40
EPW = E // (NC * NS)
NCHD = EPW // CHD
NZD = N // CHD
ZKD = (NZD + NS - 1) // NS


def _deg_body(dst_hbm, ones_hbm, zeros_hbm, out_hbm, *rest):
    dstv = rest[0:4]
    onesv = rest[4]
    rowsv = rest[5]
    deg_sh = rest[6]
    sem_d = rest[7:11]
    sem_s = rest[11:15]

    c = lax.axis_index("c")
    s = lax.axis_index("s")

    pltpu.sync_copy(zeros_hbm, rowsv)
    for k in range(ZKD):
        cid = s + k * NS

        @pl.when(cid < NZD)
        def _():
            pltpu.sync_copy(rowsv, deg_sh.at[pl.ds(cid * CHD, CHD)])
    pltpu.sync_copy(ones_hbm, onesv)
    plsc.subcore_barrier()

    w = c * NS + s

    def fire_dst(k, b):
        base = pl.multiple_of(w * EPW + k * CHD, 8)
        pltpu.async_copy(dst_hbm.at[pl.ds(base, CHD)], dstv[b], sem_d[b])

    def wait_dst(b):
        pltpu.make_async_copy(dst_hbm.at[pl.ds(0, CHD)], dstv[b],
                              sem_d[b]).wait()

    def fire_scatter(b):
        pltpu.async_copy(onesv, deg_sh.at[dstv[b]], sem_s[b], add=True)

    def wait_scatter(b):
        pltpu.make_async_copy(onesv, deg_sh.at[dstv[b]], sem_s[b]).wait()

    fire_dst(0, 0)
    fire_dst(1, 1)

    @pl.loop(0, NCHD, step=DEPTH)
    def _(g):
        for b in range(DEPTH):
            k = g + b
            b2 = (b + 2) % DEPTH

            @pl.when(k < NCHD)
            def _():
                @pl.when(k + 2 < NCHD)
                def _():
                    @pl.when(k >= 2)
                    def _():
                        wait_scatter(b2)
                    fire_dst(k + 2, b2)
                wait_dst(b)
                fire_scatter(b)

    for j in range(DEPTH):
        wait_scatter((NCHD - DEPTH + j) % DEPTH)
    plsc.subcore_barrier()

    for k in range(ZKD):
        cid = s + k * NS

        @pl.when(cid < NZD)
        def _():
            pltpu.sync_copy(deg_sh.at[pl.ds(cid * CHD, CHD)], rowsv)
            pltpu.sync_copy(rowsv, out_hbm.at[pl.ds(c * N + cid * CHD, CHD)])


def _degree(dst):
    mesh = plsc.VectorSubcoreMesh(core_axis_name="c", subcore_axis_name="s")
    ones = jnp.ones((CHD, DH), jnp.float32)
    zeros = jnp.zeros((CHD, DH), jnp.float32)
    kern = pl.kernel(
        _deg_body,
        out_type=jax.ShapeDtypeStruct((2 * N, DH), jnp.float32),
        mesh=mesh,
        scratch_types=(
            [pltpu.VMEM((CHD,), jnp.int32)] * DEPTH
            + [pltpu.VMEM((CHD, DH), jnp.float32)]
            + [pltpu.VMEM((CHD, DH), jnp.float32)]
            + [pltpu.VMEM_SHARED((N, DH), jnp.float32)]
            + [pltpu.SemaphoreType.DMA] * (2 * DEPTH)
        ),
    )
    return kern(dst, ones, zeros)


def _mm_split(x, w):
    def body(x_ref, w_ref, o_ref):
        xv = x_ref[...]
        o_ref[0:N, :] = jnp.dot(xv, w_ref[:, 0:DH],
                                preferred_element_type=jnp.float32)
        o_ref[N:2 * N, :] = jnp.dot(xv, w_ref[:, DH:D],
                                    preferred_element_type=jnp.float32)
        o_ref[2 * N:, :] = jnp.zeros((8, DH), jnp.float32)
    return pl.pallas_call(
        body,
        out_shape=jax.ShapeDtypeStruct((2 * N + 8, DH), jnp.float32))(x, w)


def _combine_parts(x_ref, s_ref, d_ref, w_ref, b_ref, relu):
    a = jnp.dot(x_ref[...], w_ref[...],
                preferred_element_type=jnp.float32) + b_ref[...]
    deg = d_ref[0:N, 0:1] + d_ref[N:2 * N, 0:1]
    inv = 1.0 / jnp.maximum(deg, 1.0)
    lo = a[:, 0:DH] + s_ref[0:N, :] * inv
    hi = a[:, DH:D] + s_ref[N:2 * N, :] * inv
    if relu:
        lo = jnp.maximum(lo, 0.0)
        hi = jnp.maximum(hi, 0.0)
    return lo, hi


def _combine_relu(x, s_full, deg_full, w_self, b):
    def body(x_ref, s_ref, d_ref, w_ref, b_ref, o_ref):
        lo, hi = _combine_parts(x_ref, s_ref, d_ref, w_ref, b_ref, True)
        o_ref[:, 0:DH] = lo
        o_ref[:, DH:D] = hi
    return pl.pallas_call(
        body, out_shape=jax.ShapeDtypeStruct((N, D), jnp.float32))(
            x, s_full, deg_full, w_self, b.reshape(1, D))


def _combine(x, s_full, deg_full, w_self, b):
    def body(x_ref, s_ref, d_ref, w_ref, b_ref, o_ref):
        lo, hi = _combine_parts(x_ref, s_ref, d_ref, w_ref, b_ref, False)
        o_ref[:, 0:DH] = lo
        o_ref[:, DH:D] = hi
    return pl.pallas_call(
        body, out_shape=jax.ShapeDtypeStruct((N, D), jnp.float32))(
            x, s_full, deg_full, w_self, b.reshape(1, D))


def kernel(in_feat, edge_index, W_self1, W_neigh1, b1, W_self2, W_neigh2, b2):
    src = edge_index[0].astype(jnp.int32)
    dst = edge_index[1].astype(jnp.int32)

    src2 = src.reshape(NS, E // NS)
    idx_pad = jnp.full((NS, PAD), ZROW, jnp.int32)
    gidx = jnp.concatenate([
        jnp.concatenate([src2, idx_pad], axis=1).reshape(-1),
        jnp.concatenate([src2 + N, idx_pad], axis=1).reshape(-1)])
    dstp = jnp.concatenate([dst.reshape(NS, E // NS),
                            jnp.zeros((NS, PAD), jnp.int32)],
                           axis=1).reshape(-1)

    deg_full = _degree(dst)
    z1 = _mm_split(in_feat, W_neigh1)
    s1 = _segment_sum(gidx, dstp, z1)
    h = _combine_relu(in_feat, s1, deg_full, W_self1, b1)
    z2 = _mm_split(h, W_neigh2)
    s2 = _segment_sum(gidx, dstp, z2)
    return _combine(h, s2, deg_full, W_self2, b2)


# speedup vs baseline: 1.5706x; 1.5706x over previous
import jax
import jax.numpy as jnp
from jax import lax
from jax.experimental import pallas as pl
from jax.experimental.pallas import tpu as pltpu
from jax.experimental.pallas import tpu_sc as plsc

N = 10000
E = 160000
D = 256
DH = 128
NC = 2
NS = 16
CH = 80
EPT = 10000
PAD = EPT - E // NS
EPAD = EPT * NS
NCHUNK = EPT // CH
ZROW = 2 * N
ACH = 80
NZ = N // ACH
ZK = (NZ + NS - 1) // NS


DEPTH = 4


def _segsum_body(gidx_hbm, dst_hbm, z_hbm, zeros_hbm, out_hbm, *rest):
    idxv = rest[0:4]
    dstv = rest[4:8]
    rowsv = rest[8:12]
    acc_sh = rest[12]
    sem_i = rest[13:17]
    sem_d = rest[17:21]
    sem_g = rest[21:25]
    sem_s = rest[25:29]

    c = lax.axis_index("c")
    s = lax.axis_index("s")

    stgv = rowsv[0].at[pl.ds(0, ACH)]
    pltpu.sync_copy(zeros_hbm, stgv)
    for k in range(ZK):
        cid = s + k * NS

        @pl.when(cid < NZ)
        def _():
            pltpu.sync_copy(stgv, acc_sh.at[pl.ds(cid * ACH, ACH)])
    plsc.subcore_barrier()

    def fire_idx(k, b):
        base = pl.multiple_of(s * EPT + k * CH, 8)
        pltpu.async_copy(gidx_hbm.at[pl.ds(c * EPAD + base, CH)], idxv[b],
                         sem_i[b])
        pltpu.async_copy(dst_hbm.at[pl.ds(base, CH)], dstv[b], sem_d[b])

    def wait_idx(b):
        pltpu.make_async_copy(gidx_hbm.at[pl.ds(0, CH)], idxv[b],
                              sem_i[b]).wait()
        pltpu.make_async_copy(dst_hbm.at[pl.ds(0, CH)], dstv[b],
                              sem_d[b]).wait()

    def fire_gather(b):
        pltpu.async_copy(z_hbm.at[idxv[b]], rowsv[b], sem_g[b])

    def wait_gather(b):
        pltpu.make_async_copy(z_hbm.at[idxv[b]], rowsv[b], sem_g[b]).wait()

    def fire_scatter(b):
        pltpu.async_copy(rowsv[b], acc_sh.at[dstv[b]], sem_s[b], add=True)

    def wait_scatter(b):
        pltpu.make_async_copy(rowsv[b], acc_sh.at[dstv[b]],
                              sem_s[b]).wait()

    fire_idx(0, 0)
    fire_idx(1, 1)
    wait_idx(0)
    fire_gather(0)

    @pl.loop(0, NCHUNK, step=DEPTH)
    def _(g):
        for b in range(DEPTH):
            k = g + b
            b1 = (b + 1) % DEPTH
            b2 = (b + 2) % DEPTH

            @pl.when(k < NCHUNK)
            def _():
                @pl.when(k + 2 < NCHUNK)
                def _():
                    @pl.when(k >= 2)
                    def _():
                        wait_scatter(b2)
                    fire_idx(k + 2, b2)

                @pl.when(k + 1 < NCHUNK)
                def _():
                    wait_idx(b1)
                    fire_gather(b1)
                wait_gather(b)
                fire_scatter(b)

    for j in range(DEPTH):
        wait_scatter((NCHUNK - DEPTH + j) % DEPTH)
    plsc.subcore_barrier()

    for k in range(ZK):
        cid = s + k * NS

        @pl.when(cid < NZ)
        def _():
            pltpu.sync_copy(acc_sh.at[pl.ds(cid * ACH, ACH)], stgv)
            pltpu.sync_copy(stgv, out_hbm.at[pl.ds(c * N + cid * ACH, ACH)])


def _segment_sum(gidx, dstp, z_pad):
    mesh = plsc.VectorSubcoreMesh(core_axis_name="c", subcore_axis_name="s")
    zeros = jnp.zeros((ACH, DH), jnp.float32)
    kern = pl.kernel(
        _segsum_body,
        out_type=jax.ShapeDtypeStruct((2 * N, DH), jnp.float32),
        mesh=mesh,
        scratch_types=(
            [pltpu.VMEM((CH,), jnp.int32)] * DEPTH
            + [pltpu.VMEM((CH,), jnp.int32)] * DEPTH
            + [pltpu.VMEM((CH, DH), jnp.float32)] * DEPTH
            + [pltpu.VMEM_SHARED((N, DH), jnp.float32)]
            + [pltpu.SemaphoreType.DMA] * (4 * DEPTH)
        ),
    )
    return kern(gidx, dstp, z_pad, zeros)


CHD = 40
EPW = E // (NC * NS)
NCHD = EPW // CHD
NZD = N // CHD
ZKD = (NZD + NS - 1) // NS


def _deg_body(dst_hbm, ones_hbm, zeros_hbm, out_hbm, *rest):
    dstv = rest[0:4]
    onesv = rest[4]
    rowsv = rest[5]
    deg_sh = rest[6]
    sem_d = rest[7:11]
    sem_s = rest[11:15]

    c = lax.axis_index("c")
    s = lax.axis_index("s")

    pltpu.sync_copy(zeros_hbm, rowsv)
    for k in range(ZKD):
        cid = s + k * NS

        @pl.when(cid < NZD)
        def _():
            pltpu.sync_copy(rowsv, deg_sh.at[pl.ds(cid * CHD, CHD)])
    pltpu.sync_copy(ones_hbm, onesv)
    plsc.subcore_barrier()

    w = c * NS + s

    def fire_dst(k, b):
        base = pl.multiple_of(w * EPW + k * CHD, 8)
        pltpu.async_copy(dst_hbm.at[pl.ds(base, CHD)], dstv[b], sem_d[b])

    def wait_dst(b):
        pltpu.make_async_copy(dst_hbm.at[pl.ds(0, CHD)], dstv[b],
                              sem_d[b]).wait()

    def fire_scatter(b):
        pltpu.async_copy(onesv, deg_sh.at[dstv[b]], sem_s[b], add=True)

    def wait_scatter(b):
        pltpu.make_async_copy(onesv, deg_sh.at[dstv[b]], sem_s[b]).wait()

    fire_dst(0, 0)
    fire_dst(1, 1)

    @pl.loop(0, NCHD, step=DEPTH)
    def _(g):
        for b in range(DEPTH):
            k = g + b
            b2 = (b + 2) % DEPTH

            @pl.when(k < NCHD)
            def _():
                @pl.when(k + 2 < NCHD)
                def _():
                    @pl.when(k >= 2)
                    def _():
                        wait_scatter(b2)
                    fire_dst(k + 2, b2)
                wait_dst(b)
                fire_scatter(b)

    for j in range(DEPTH):
        wait_scatter((NCHD - DEPTH + j) % DEPTH)
    plsc.subcore_barrier()

    for k in range(ZKD):
        cid = s + k * NS

        @pl.when(cid < NZD)
        def _():
            pltpu.sync_copy(deg_sh.at[pl.ds(cid * CHD, CHD)], rowsv)
            pltpu.sync_copy(rowsv, out_hbm.at[pl.ds(c * N + cid * CHD, CHD)])


def _degree(dst):
    mesh = plsc.VectorSubcoreMesh(core_axis_name="c", subcore_axis_name="s")
    ones = jnp.ones((CHD, DH), jnp.float32)
    zeros = jnp.zeros((CHD, DH), jnp.float32)
    kern = pl.kernel(
        _deg_body,
        out_type=jax.ShapeDtypeStruct((2 * N, DH), jnp.float32),
        mesh=mesh,
        scratch_types=(
            [pltpu.VMEM((CHD,), jnp.int32)] * DEPTH
            + [pltpu.VMEM((CHD, DH), jnp.float32)]
            + [pltpu.VMEM((CHD, DH), jnp.float32)]
            + [pltpu.VMEM_SHARED((N, DH), jnp.float32)]
            + [pltpu.SemaphoreType.DMA] * (2 * DEPTH)
        ),
    )
    return kern(dst, ones, zeros)


def _mm_split(x, w):
    def body(x_ref, w_ref, o_ref):
        xv = x_ref[...]
        o_ref[0:N, :] = jnp.dot(xv, w_ref[:, 0:DH],
                                preferred_element_type=jnp.float32)
        o_ref[N:2 * N, :] = jnp.dot(xv, w_ref[:, DH:D],
                                    preferred_element_type=jnp.float32)
        o_ref[2 * N:, :] = jnp.zeros((8, DH), jnp.float32)
    return pl.pallas_call(
        body,
        out_shape=jax.ShapeDtypeStruct((2 * N + 8, DH), jnp.float32))(x, w)


def _combine_parts(x_ref, s_ref, d_ref, w_ref, b_ref, relu):
    a = jnp.dot(x_ref[...], w_ref[...],
                preferred_element_type=jnp.float32) + b_ref[...]
    deg = d_ref[0:N, 0:1] + d_ref[N:2 * N, 0:1]
    inv = 1.0 / jnp.maximum(deg, 1.0)
    lo = a[:, 0:DH] + s_ref[0:N, :] * inv
    hi = a[:, DH:D] + s_ref[N:2 * N, :] * inv
    if relu:
        lo = jnp.maximum(lo, 0.0)
        hi = jnp.maximum(hi, 0.0)
    return lo, hi


def _combine_relu(x, s_full, deg_full, w_self, b):
    def body(x_ref, s_ref, d_ref, w_ref, b_ref, o_ref):
        lo, hi = _combine_parts(x_ref, s_ref, d_ref, w_ref, b_ref, True)
        o_ref[:, 0:DH] = lo
        o_ref[:, DH:D] = hi
    return pl.pallas_call(
        body, out_shape=jax.ShapeDtypeStruct((N, D), jnp.float32))(
            x, s_full, deg_full, w_self, b.reshape(1, D))


def _combine(x, s_full, deg_full, w_self, b):
    def body(x_ref, s_ref, d_ref, w_ref, b_ref, o_ref):
        lo, hi = _combine_parts(x_ref, s_ref, d_ref, w_ref, b_ref, False)
        o_ref[:, 0:DH] = lo
        o_ref[:, DH:D] = hi
    return pl.pallas_call(
        body, out_shape=jax.ShapeDtypeStruct((N, D), jnp.float32))(
            x, s_full, deg_full, w_self, b.reshape(1, D))


def kernel(in_feat, edge_index, W_self1, W_neigh1, b1, W_self2, W_neigh2, b2):
    src = edge_index[0].astype(jnp.int32)
    dst = edge_index[1].astype(jnp.int32)

    if PAD:
        src2 = src.reshape(NS, E // NS)
        idx_pad = jnp.full((NS, PAD), ZROW, jnp.int32)
        gidx = jnp.concatenate([
            jnp.concatenate([src2, idx_pad], axis=1).reshape(-1),
            jnp.concatenate([src2 + N, idx_pad], axis=1).reshape(-1)])
        dstp = jnp.concatenate([dst.reshape(NS, E // NS),
                                jnp.zeros((NS, PAD), jnp.int32)],
                               axis=1).reshape(-1)
    else:
        gidx = jnp.concatenate([src, src + N])
        dstp = dst

    deg_full = _degree(dst)
    z1 = _mm_split(in_feat, W_neigh1)
    s1 = _segment_sum(gidx, dstp, z1)
    h = _combine_relu(in_feat, s1, deg_full, W_self1, b1)
    z2 = _mm_split(h, W_neigh2)
    s2 = _segment_sum(gidx, dstp, z2)
    return _combine(h, s2, deg_full, W_self2, b2)
